# baseline (speedup 1.0000x reference)
import jax
import jax.numpy as jnp
from jax.experimental import pallas as pl
from jax.experimental.pallas import tpu as pltpu

_INPUT_LEN = 200
_EMBED_DIM = 64
_BATCH = 4096
_TR = 256
_NB = _BATCH // _TR


def _bcast_body(pos_ref, out_ref, tile_ref, sem):
    tile_ref[...] = jnp.broadcast_to(pos_ref[...][None], tile_ref.shape)
    copies = [
        pltpu.make_async_copy(tile_ref, out_ref.at[pl.ds(j * _TR, _TR)], sem)
        for j in range(_NB)
    ]
    for c in copies:
        c.start()
    for c in copies:
        c.wait()


def kernel(x, pos_table):
    del x
    return pl.pallas_call(
        _bcast_body,
        in_specs=[pl.BlockSpec((_INPUT_LEN, _EMBED_DIM), lambda: (0, 0))],
        out_specs=pl.BlockSpec(memory_space=pl.ANY),
        out_shape=jax.ShapeDtypeStruct((_BATCH, _INPUT_LEN, _EMBED_DIM), jnp.float32),
        scratch_shapes=[
            pltpu.VMEM((_TR, _INPUT_LEN, _EMBED_DIM), jnp.float32),
            pltpu.SemaphoreType.DMA,
        ],
    )(pos_table)
